# Initial kernel scaffold; baseline (speedup 1.0000x reference)
import jax
import jax.numpy as jnp
from jax.experimental import pallas as pl


def kernel(memory, last_update, user_indices, item_embedding, interaction_features, timestamps, w1, b1, w2, b2, w_ih, w_hh, b_ih, b_hh):
    raise NotImplementedError("write your pallas kernel here")


# speedup vs baseline: 1.8783x; 1.8783x over previous
import functools

import jax
import jax.numpy as jnp
from jax import lax
from jax.experimental import pallas as pl
from jax.experimental.pallas import tpu as pltpu
from jax.experimental.pallas import tpu_sc as plsc

NC = 2
NS = 16
NW = NC * NS
CH = 64


def _mesh():
    return plsc.VectorSubcoreMesh(core_axis_name="c", subcore_axis_name="s")


def _sc_gather(mem, idx3, nch):
    d = mem.shape[1]
    b = NW * nch * CH

    @functools.partial(
        pl.kernel,
        mesh=_mesh(),
        out_type=jax.ShapeDtypeStruct((b, d), jnp.float32),
        scratch_types=[
            pltpu.VMEM((nch, CH), jnp.int32),
            pltpu.VMEM((CH, d), jnp.float32),
            pltpu.VMEM((CH, d), jnp.float32),
            pltpu.SemaphoreType.DMA,
            pltpu.SemaphoreType.DMA,
        ],
    )
    def k(mem_hbm, idx_hbm, out_hbm, idx_v, rows0, rows1, sem0, sem1):
        wid = lax.axis_index("s") * NC + lax.axis_index("c")
        base = wid * nch * CH
        pltpu.sync_copy(idx_hbm.at[wid], idx_v)
        bufs = (rows0, rows1)
        sems = (sem0, sem1)
        pltpu.async_copy(mem_hbm.at[idx_v.at[0]], rows0, sem0)

        def body(j, _):
            slot = lax.rem(j, 2)
            nxt = lax.rem(j + 1, 2)

            @pl.when(j + 1 < nch)
            def _():
                for s in range(2):
                    @pl.when(nxt == s)
                    def _():
                        pltpu.async_copy(
                            mem_hbm.at[idx_v.at[j + 1]], bufs[s], sems[s])

            for s in range(2):
                @pl.when(slot == s)
                def _():
                    pltpu.make_async_copy(
                        mem_hbm.at[idx_v.at[j]], bufs[s], sems[s]).wait()
                    pltpu.sync_copy(
                        bufs[s], out_hbm.at[pl.ds(base + j * CH, CH)])
            return 0

        lax.fori_loop(0, nch, body, 0)

    return k(mem, idx3)


def _sc_scatter(mem_ref, last_ref, new_mem, ts, dst3, src3, nch):
    d = new_mem.shape[1]

    @functools.partial(
        pl.kernel,
        mesh=_mesh(),
        out_type=(),
        scratch_types=[
            pltpu.VMEM((nch, CH), jnp.int32),
            pltpu.VMEM((nch, CH), jnp.int32),
            pltpu.VMEM((CH, d), jnp.float32),
            pltpu.VMEM((nch, CH), jnp.float32),
            pltpu.SemaphoreType.DMA,
            pltpu.SemaphoreType.DMA,
        ],
    )
    def k(new_hbm, ts_hbm, dst_hbm, src_hbm, mem_out, last_out,
          dst_v, src_v, rows_v, ts_v, sem, sem2):
        wid = lax.axis_index("s") * NC + lax.axis_index("c")
        pltpu.sync_copy(dst_hbm.at[wid], dst_v)
        pltpu.sync_copy(src_hbm.at[wid], src_v)

        def body(j, _):
            pltpu.async_copy(ts_hbm.at[src_v.at[j]], ts_v.at[j], sem2)
            pltpu.async_copy(new_hbm.at[src_v.at[j]], rows_v, sem).wait()
            pltpu.async_copy(rows_v, mem_out.at[dst_v.at[j]], sem).wait()
            pltpu.make_async_copy(
                ts_hbm.at[src_v.at[j]], ts_v.at[j], sem2).wait()
            pltpu.async_copy(ts_v.at[j], last_out.at[dst_v.at[j]], sem2).wait()
            return 0

        lax.fori_loop(0, nch, body, 0)

    k(new_mem, ts, dst3, src3, mem_ref, last_ref)


def _dense(um, ie, ft, w1, b1, w2, b2, w_ih, w_hh, b_ih, b_hh):
    b, d = um.shape
    msg = w1.shape[0]
    tb = 512
    grid = b // tb

    w1t = w1.T
    w1a, w1b, w1c = w1t[:d], w1t[d:2 * d], w1t[2 * d:]
    w2t = w2.T
    wih = w_ih.T
    whh = w_hh.T
    b1r = b1.reshape(1, -1)
    b2r = b2.reshape(1, -1)
    bihr = b_ih.reshape(1, -1)
    bhhr = b_hh.reshape(1, -1)

    def body(um_r, ie_r, ft_r, w1a_r, w1b_r, w1c_r, w2t_r, wih_r, whh_r,
             b1_r, b2_r, bih_r, bhh_r, out_r):
        umv = um_r[:]
        f32 = jnp.float32
        x = jnp.dot(umv, w1a_r[:], preferred_element_type=f32)
        x = x + jnp.dot(ie_r[:], w1b_r[:], preferred_element_type=f32)
        x = x + jnp.dot(ft_r[:], w1c_r[:], preferred_element_type=f32)
        h1 = jnp.maximum(x + b1_r[:], 0.0)
        m = jnp.dot(h1, w2t_r[:], preferred_element_type=f32) + b2_r[:]
        gi = jnp.dot(m, wih_r[:], preferred_element_type=f32) + bih_r[:]
        gh = jnp.dot(umv, whh_r[:], preferred_element_type=f32) + bhh_r[:]
        r = jax.nn.sigmoid(gi[:, :d] + gh[:, :d])
        z = jax.nn.sigmoid(gi[:, d:2 * d] + gh[:, d:2 * d])
        n = jnp.tanh(gi[:, 2 * d:] + r * gh[:, 2 * d:])
        out_r[:] = (1.0 - z) * n + z * umv

    const = lambda shape: pl.BlockSpec(shape, lambda i: (0, 0))
    batch = lambda shape: pl.BlockSpec(shape, lambda i: (i, 0))
    return pl.pallas_call(
        body,
        grid=(grid,),
        in_specs=[
            batch((tb, d)), batch((tb, d)), batch((tb, msg)),
            const((d, msg)), const((d, msg)), const((msg, msg)),
            const((msg, msg)), const((msg, 3 * d)), const((d, 3 * d)),
            const((1, msg)), const((1, msg)),
            const((1, 3 * d)), const((1, 3 * d)),
        ],
        out_specs=batch((tb, d)),
        out_shape=jax.ShapeDtypeStruct((b, d), jnp.float32),
        compiler_params=pltpu.CompilerParams(
            dimension_semantics=("arbitrary",)),
    )(um, ie, ft, w1a, w1b, w1c, w2t, wih, whh, b1r, b2r, bihr, bhhr)


def kernel(memory, last_update, user_indices, item_embedding,
           interaction_features, timestamps,
           w1, b1, w2, b2, w_ih, w_hh, b_ih, b_hh):
    u = memory.shape[0]
    b = user_indices.shape[0]
    nch = b // (NW * CH)

    ui = user_indices.astype(jnp.int32)
    iota = jnp.arange(b, dtype=jnp.int32)
    win = jnp.full((u,), -1, jnp.int32).at[ui].max(iota)
    src = win[ui]

    idx3 = ui.reshape(NW, nch, CH)
    src3 = src.reshape(NW, nch, CH)

    user_memory = _sc_gather(memory, idx3, nch)
    new_memory = _dense(user_memory, item_embedding, interaction_features,
                        w1, b1, w2, b2, w_ih, w_hh, b_ih, b_hh)

    mem_ref = jax.new_ref(memory)
    last_ref = jax.new_ref(last_update)
    _sc_scatter(mem_ref, last_ref, new_memory, timestamps, idx3, src3, nch)
    return new_memory, mem_ref[...], last_ref[...]


# speedup vs baseline: 1.8958x; 1.0093x over previous
import functools

import jax
import jax.numpy as jnp
from jax import lax
from jax.experimental import pallas as pl
from jax.experimental.pallas import tpu as pltpu
from jax.experimental.pallas import tpu_sc as plsc

NC = 2
NS = 16
NW = NC * NS
CH = 64


def _mesh():
    return plsc.VectorSubcoreMesh(core_axis_name="c", subcore_axis_name="s")


def _sc_gather(mem, idx3, nch):
    d = mem.shape[1]
    b = NW * nch * CH

    @functools.partial(
        pl.kernel,
        mesh=_mesh(),
        out_type=jax.ShapeDtypeStruct((b, d), jnp.float32),
        scratch_types=[
            pltpu.VMEM((nch, CH), jnp.int32),
            pltpu.VMEM((CH, d), jnp.float32),
            pltpu.VMEM((CH, d), jnp.float32),
            pltpu.SemaphoreType.DMA,
            pltpu.SemaphoreType.DMA,
        ],
    )
    def k(mem_hbm, idx_hbm, out_hbm, idx_v, rows0, rows1, sem0, sem1):
        wid = lax.axis_index("s") * NC + lax.axis_index("c")
        base = wid * nch * CH
        pltpu.sync_copy(idx_hbm.at[wid], idx_v)
        bufs = (rows0, rows1)
        sems = (sem0, sem1)
        pltpu.async_copy(mem_hbm.at[idx_v.at[0]], rows0, sem0)

        def body(j, _):
            slot = lax.rem(j, 2)
            nxt = lax.rem(j + 1, 2)

            @pl.when(j + 1 < nch)
            def _():
                for s in range(2):
                    @pl.when(nxt == s)
                    def _():
                        pltpu.async_copy(
                            mem_hbm.at[idx_v.at[j + 1]], bufs[s], sems[s])

            for s in range(2):
                @pl.when(slot == s)
                def _():
                    pltpu.make_async_copy(
                        mem_hbm.at[idx_v.at[j]], bufs[s], sems[s]).wait()
                    pltpu.sync_copy(
                        bufs[s], out_hbm.at[pl.ds(base + j * CH, CH)])
            return 0

        lax.fori_loop(0, nch, body, 0)

    return k(mem, idx3)


def _sc_scatter(mem_ref, last_ref, new_mem, ts, dst3, src3, nch):
    d = new_mem.shape[1]

    @functools.partial(
        pl.kernel,
        mesh=_mesh(),
        out_type=(),
        scratch_types=[
            pltpu.VMEM((nch, CH), jnp.int32),
            pltpu.VMEM((nch, CH), jnp.int32),
            pltpu.VMEM((CH, d), jnp.float32),
            pltpu.VMEM((nch, CH), jnp.float32),
            pltpu.SemaphoreType.DMA,
            pltpu.SemaphoreType.DMA,
        ],
    )
    def k(new_hbm, ts_hbm, dst_hbm, src_hbm, mem_out, last_out,
          dst_v, src_v, rows_v, ts_v, sem, sem2):
        wid = lax.axis_index("s") * NC + lax.axis_index("c")
        pltpu.sync_copy(dst_hbm.at[wid], dst_v)
        pltpu.sync_copy(src_hbm.at[wid], src_v)

        def body(j, _):
            pltpu.async_copy(ts_hbm.at[src_v.at[j]], ts_v.at[j], sem2)
            pltpu.async_copy(new_hbm.at[src_v.at[j]], rows_v, sem).wait()
            pltpu.async_copy(rows_v, mem_out.at[dst_v.at[j]], sem).wait()
            pltpu.make_async_copy(
                ts_hbm.at[src_v.at[j]], ts_v.at[j], sem2).wait()
            pltpu.async_copy(ts_v.at[j], last_out.at[dst_v.at[j]], sem2).wait()
            return 0

        lax.fori_loop(0, nch, body, 0)

    k(new_mem, ts, dst3, src3, mem_ref, last_ref)


def _dense(um, ie, ft, w1, b1, w2, b2, w_ih, w_hh, b_ih, b_hh):
    b, d = um.shape
    msg = w1.shape[0]
    tb = 512
    grid = b // tb

    bf16 = jnp.bfloat16
    w1t = w1.T.astype(bf16)
    w1a, w1b, w1c = w1t[:d], w1t[d:2 * d], w1t[2 * d:]
    w2t = w2.T.astype(bf16)
    wih = w_ih.T.astype(bf16)
    whh = w_hh.T.astype(bf16)
    b1r = b1.reshape(1, -1)
    b2r = b2.reshape(1, -1)
    bihr = b_ih.reshape(1, -1)
    bhhr = b_hh.reshape(1, -1)

    def body(um_r, ie_r, ft_r, w1a_r, w1b_r, w1c_r, w2t_r, wih_r, whh_r,
             b1_r, b2_r, bih_r, bhh_r, out_r):
        umv = um_r[:]
        f32 = jnp.float32
        bf = jnp.bfloat16
        umb = umv.astype(bf)
        x = jnp.dot(umb, w1a_r[:], preferred_element_type=f32)
        x = x + jnp.dot(ie_r[:].astype(bf), w1b_r[:], preferred_element_type=f32)
        x = x + jnp.dot(ft_r[:].astype(bf), w1c_r[:], preferred_element_type=f32)
        h1 = jnp.maximum(x + b1_r[:], 0.0)
        m = jnp.dot(h1.astype(bf), w2t_r[:], preferred_element_type=f32) + b2_r[:]
        gi = jnp.dot(m.astype(bf), wih_r[:], preferred_element_type=f32) + bih_r[:]
        gh = jnp.dot(umb, whh_r[:], preferred_element_type=f32) + bhh_r[:]
        r = jax.nn.sigmoid(gi[:, :d] + gh[:, :d])
        z = jax.nn.sigmoid(gi[:, d:2 * d] + gh[:, d:2 * d])
        n = jnp.tanh(gi[:, 2 * d:] + r * gh[:, 2 * d:])
        out_r[:] = (1.0 - z) * n + z * umv

    const = lambda shape: pl.BlockSpec(shape, lambda i: (0, 0))
    batch = lambda shape: pl.BlockSpec(shape, lambda i: (i, 0))
    return pl.pallas_call(
        body,
        grid=(grid,),
        in_specs=[
            batch((tb, d)), batch((tb, d)), batch((tb, msg)),
            const((d, msg)), const((d, msg)), const((msg, msg)),
            const((msg, msg)), const((msg, 3 * d)), const((d, 3 * d)),
            const((1, msg)), const((1, msg)),
            const((1, 3 * d)), const((1, 3 * d)),
        ],
        out_specs=batch((tb, d)),
        out_shape=jax.ShapeDtypeStruct((b, d), jnp.float32),
        compiler_params=pltpu.CompilerParams(
            dimension_semantics=("arbitrary",)),
    )(um, ie, ft, w1a, w1b, w1c, w2t, wih, whh, b1r, b2r, bihr, bhhr)


def kernel(memory, last_update, user_indices, item_embedding,
           interaction_features, timestamps,
           w1, b1, w2, b2, w_ih, w_hh, b_ih, b_hh):
    u = memory.shape[0]
    b = user_indices.shape[0]
    nch = b // (NW * CH)

    ui = user_indices.astype(jnp.int32)
    iota = jnp.arange(b, dtype=jnp.int32)
    win = jnp.full((u,), -1, jnp.int32).at[ui].max(iota)
    src = win[ui]

    idx3 = ui.reshape(NW, nch, CH)
    src3 = src.reshape(NW, nch, CH)

    user_memory = _sc_gather(memory, idx3, nch)
    new_memory = _dense(user_memory, item_embedding, interaction_features,
                        w1, b1, w2, b2, w_ih, w_hh, b_ih, b_hh)

    mem_ref = jax.new_ref(memory)
    last_ref = jax.new_ref(last_update)
    _sc_scatter(mem_ref, last_ref, new_memory, timestamps, idx3, src3, nch)
    return new_memory, mem_ref[...], last_ref[...]
